# baseline (speedup 1.0000x reference)
import functools

import jax
import jax.numpy as jnp
from jax import lax
from jax.experimental import pallas as pl
from jax.experimental.pallas import tpu as pltpu
from jax.experimental.pallas import tpu_sc as plsc

N_NODES = 10000
N_EDGES = 320000
D = 128
DE = 16

NC = 2
NS = 16
NW = NC * NS
EPW = N_EDGES // NW
EH = N_EDGES // 2
EPWH = EH // NW
CH = 40
NCHUNK = EPWH // CH
NP = 10240
RPT = NP // NS
NSUB = 1
NVREG = EPW // 16

_SC_MESH = plsc.VectorSubcoreMesh(
    core_axis_name="c", subcore_axis_name="s", num_cores=NC, num_subcores=NS)


def _gather_body(x_hbm, row_hbm, col_hbm, gr_hbm, gc_hbm,
                 row_v, col_v, gr_v, gc_v, sem_r, sem_c):
    wid = lax.axis_index("s") * NC + lax.axis_index("c")
    pltpu.sync_copy(row_hbm.at[wid], row_v)
    pltpu.sync_copy(col_hbm.at[wid], col_v)

    def body(i, carry):
        base = wid * EPWH + i * CH
        cpr = pltpu.async_copy(x_hbm.at[row_v.at[i]], gr_v, sem_r)
        cpc = pltpu.async_copy(x_hbm.at[col_v.at[i]], gc_v, sem_c)
        cpr.wait()
        cpc.wait()
        pltpu.sync_copy(gr_v, gr_hbm.at[pl.ds(base, CH)])
        pltpu.sync_copy(gc_v, gc_hbm.at[pl.ds(base, CH)])
        return carry

    lax.fori_loop(0, NCHUNK, body, 0)


@functools.partial(
    pl.kernel,
    out_type=[
        jax.ShapeDtypeStruct((EH, D), jnp.float32),
        jax.ShapeDtypeStruct((EH, D), jnp.float32),
    ],
    mesh=_SC_MESH,
    scratch_types=[
        pltpu.VMEM((NCHUNK, CH), jnp.int32),
        pltpu.VMEM((NCHUNK, CH), jnp.int32),
        pltpu.VMEM((CH, D), jnp.float32),
        pltpu.VMEM((CH, D), jnp.float32),
        pltpu.SemaphoreType.DMA,
        pltpu.SemaphoreType.DMA,
    ],
)
def _gather(x, row3, col3, gr, gc, *scratch):
    _gather_body(x, row3, col3, gr, gc, *scratch)


def _edge_body(gr_ref, gc_ref, ea_ref, w1a_ref, w1b_ref, w1c_ref, b1_ref,
               w2_ref, b2_ref, nw1a_ref, nw1b_ref, nb1_ref,
               nea_ref, hn_ref):
    f32 = jnp.float32
    bf16 = jnp.bfloat16
    gr = gr_ref[...].astype(bf16)
    gc = gc_ref[...].astype(bf16)
    ab = (jnp.dot(gr, w1a_ref[...], preferred_element_type=f32)
          + jnp.dot(gc, w1b_ref[...], preferred_element_type=f32)
          + jnp.dot(ea_ref[...], w1c_ref[...], preferred_element_type=f32)
          + b1_ref[...])
    he = jnp.maximum(ab, 0.0).astype(bf16)
    nea = jnp.dot(he, w2_ref[...], preferred_element_type=f32) + b2_ref[...]
    nea_ref[...] = nea
    hn_ref[...] = jnp.maximum(
        jnp.dot(gr, nw1a_ref[...], preferred_element_type=f32)
        + jnp.dot(nea.astype(bf16), nw1b_ref[...], preferred_element_type=f32)
        + nb1_ref[...], 0.0)


def _edge(gr, gc, ea, w1a, w1b, w1c, b1, w2, b2, nw1a, nw1b, nb1):
    blk = 4000

    def full(shape):
        return pl.BlockSpec(shape, lambda i: (0, 0))

    return pl.pallas_call(
        _edge_body,
        grid=(EH // blk,),
        in_specs=[
            pl.BlockSpec((blk, D), lambda i: (i, 0)),
            pl.BlockSpec((blk, D), lambda i: (i, 0)),
            pl.BlockSpec((blk, DE), lambda i: (i, 0)),
            full((D, DE)),
            full((D, DE)),
            full((DE, DE)),
            full((1, DE)),
            full((DE, DE)),
            full((1, DE)),
            full((D, D)),
            full((DE, D)),
            full((1, D)),
        ],
        out_specs=[
            pl.BlockSpec((blk, DE), lambda i: (i, 0)),
            pl.BlockSpec((blk, D), lambda i: (i, 0)),
        ],
        out_shape=[
            jax.ShapeDtypeStruct((EH, DE), jnp.float32),
            jax.ShapeDtypeStruct((EH, D), jnp.float32),
        ],
    )(gr, gc, ea, w1a, w1b, w1c, b1, w2, b2, nw1a, nw1b, nb1)


def _scatter_common(hn_hbm, col_hbm, zacc_hbm, part_hbm, col_v, hn_v, acc):
    c_id = lax.axis_index("c")
    s_id = lax.axis_index("s")
    wid = s_id * NC + c_id
    rbase = s_id * RPT
    pltpu.sync_copy(zacc_hbm.at[pl.ds(rbase, RPT)], acc.at[pl.ds(rbase, RPT)])
    pltpu.sync_copy(col_hbm.at[wid], col_v)
    plsc.subcore_barrier()

    def body(i, carry):
        base = wid * EPWH + i * CH
        pltpu.sync_copy(hn_hbm.at[pl.ds(base, CH)], hn_v)
        pltpu.sync_copy(hn_v, acc.at[col_v.at[i]], add=True)
        return carry

    lax.fori_loop(0, NCHUNK, body, 0)


def _scatter_finish(part_hbm, acc):
    c_id = lax.axis_index("c")
    s_id = lax.axis_index("s")
    rbase = s_id * RPT
    plsc.subcore_barrier()
    pltpu.sync_copy(acc.at[pl.ds(rbase, RPT)],
                    part_hbm.at[c_id].at[pl.ds(rbase, RPT)])


def _histogram(col2_v, cnt8_v):
    lane = lax.iota(jnp.int32, 16)
    ioff = lax.rem(lane, NSUB) * NP
    group = lane // NSUB
    masks = [group == g for g in range(16 // NSUB)]
    ones16 = jnp.full((16,), 1.0, jnp.float32)

    def cbody(k, carry):
        idx = col2_v[pl.ds(k * 16, 16)] + ioff
        for m in masks:
            plsc.addupdate_scatter(cnt8_v, [idx], ones16, mask=m)
        return carry

    lax.fori_loop(0, NVREG, cbody, 0)


@functools.partial(
    pl.kernel,
    out_type=[
        jax.ShapeDtypeStruct((NC, NP, D), jnp.float32),
        jax.ShapeDtypeStruct((NW, NSUB * NP), jnp.float32),
    ],
    mesh=_SC_MESH,
    scratch_types=[
        pltpu.VMEM((NCHUNK, CH), jnp.int32),
        pltpu.VMEM((EPW,), jnp.int32),
        pltpu.VMEM((CH, D), jnp.float32),
        pltpu.VMEM((NSUB * NP,), jnp.float32),
        pltpu.VMEM_SHARED((NP, D), jnp.float32),
    ],
    compiler_params=pltpu.CompilerParams(needs_layout_passes=False),
)
def _scatter_hist(hn, col3, col2, zacc, zcnt, part, cnt,
                  col_v, col2_v, hn_v, cnt8_v, acc):
    wid = lax.axis_index("s") * NC + lax.axis_index("c")
    pltpu.sync_copy(zcnt_hbm := zcnt, cnt8_v)
    pltpu.sync_copy(col2.at[wid], col2_v)
    _scatter_common(hn, col3, zacc, part, col_v, hn_v, acc)
    _histogram(col2_v, cnt8_v)
    _scatter_finish(part, acc)
    pltpu.sync_copy(cnt8_v, cnt.at[wid])


@functools.partial(
    pl.kernel,
    out_type=jax.ShapeDtypeStruct((NC, NP, D), jnp.float32),
    mesh=_SC_MESH,
    scratch_types=[
        pltpu.VMEM((NCHUNK, CH), jnp.int32),
        pltpu.VMEM((CH, D), jnp.float32),
        pltpu.VMEM_SHARED((NP, D), jnp.float32),
    ],
    compiler_params=pltpu.CompilerParams(needs_layout_passes=False),
)
def _scatter_plain(hn, col3, zacc, part, col_v, hn_v, acc):
    _scatter_common(hn, col3, zacc, part, col_v, hn_v, acc)
    _scatter_finish(part, acc)


def _post_body(p0_ref, p1_ref, p2_ref, p3_ref, c_ref, ones_ref, w_ref,
               nb2_ref, out_ref):
    sums = (p0_ref[...] + p1_ref[...]) + (p2_ref[...] + p3_ref[...])
    cnt = jnp.dot(c_ref[...], ones_ref[...],
                  preferred_element_type=jnp.float32)
    denom = jnp.maximum(cnt, 1.0)
    out_ref[...] = jnp.maximum(
        (jnp.dot(sums, w_ref[...], preferred_element_type=jnp.float32)
         + nb2_ref[...] * cnt) / denom, 0.0)


def _post(p0, p1, p2, p3, cnt_t, ones, w, nb2):
    blk = 1280
    nsh = NW * NSUB
    return pl.pallas_call(
        _post_body,
        grid=(NP // blk,),
        in_specs=[
            pl.BlockSpec((blk, D), lambda i: (i, 0)),
            pl.BlockSpec((blk, D), lambda i: (i, 0)),
            pl.BlockSpec((blk, D), lambda i: (i, 0)),
            pl.BlockSpec((blk, D), lambda i: (i, 0)),
            pl.BlockSpec((blk, nsh), lambda i: (i, 0)),
            pl.BlockSpec((nsh, 1), lambda i: (0, 0)),
            pl.BlockSpec((D, D), lambda i: (0, 0)),
            pl.BlockSpec((1, D), lambda i: (0, 0)),
        ],
        out_specs=pl.BlockSpec((blk, D), lambda i: (i, 0)),
        out_shape=jax.ShapeDtypeStruct((NP, D), jnp.float32),
    )(p0, p1, p2, p3, cnt_t, ones, w, nb2)


def kernel(x, edge_index, edge_attr, eW1, eb1, eW2, eb2, nW1, nb1, nW2, nb2):
    row = edge_index[0].astype(jnp.int32)
    col = edge_index[1].astype(jnp.int32)
    rowA = row[:EH].reshape(NW, NCHUNK, CH)
    colA = col[:EH].reshape(NW, NCHUNK, CH)
    rowB = row[EH:].reshape(NW, NCHUNK, CH)
    colB = col[EH:].reshape(NW, NCHUNK, CH)
    col2 = col.reshape(NW, EPW)

    bf16 = jnp.bfloat16
    grA, gcA = _gather(x, rowA, colA)
    grB, gcB = _gather(x, rowB, colB)

    ws = (eW1[:D].astype(bf16), eW1[D:2 * D].astype(bf16),
          eW1[2 * D:].astype(bf16), eb1.reshape(1, DE),
          eW2.astype(bf16), eb2.reshape(1, DE),
          nW1[:D].astype(bf16), nW1[D:].astype(bf16), nb1.reshape(1, D))
    eab = edge_attr.astype(bf16)
    neaA, hnA = _edge(grA, gcA, eab[:EH], *ws)
    neaB, hnB = _edge(grB, gcB, eab[EH:], *ws)

    zacc = jnp.zeros((NP, D), jnp.float32)
    zcnt = jnp.zeros((NSUB * NP,), jnp.float32)
    partA, cnt = _scatter_hist(hnA, colA, col2, zacc, zcnt)
    partB = _scatter_plain(hnB, colB, zacc)
    cnt_t = cnt.reshape(NW * NSUB, NP).T
    ones = jnp.ones((NW * NSUB, 1), jnp.float32)
    out = _post(partA[0], partA[1], partB[0], partB[1], cnt_t, ones,
                nW2, nb2.reshape(1, D))
    return out[:N_NODES], jnp.concatenate([neaA, neaB], axis=0)


# speedup vs baseline: 1.1532x; 1.1532x over previous
import functools

import jax
import jax.numpy as jnp
from jax import lax
from jax.experimental import pallas as pl
from jax.experimental.pallas import tpu as pltpu
from jax.experimental.pallas import tpu_sc as plsc

N_NODES = 10000
N_EDGES = 320000
D = 128
DE = 16

NC = 2
NS = 16
NW = NC * NS
EPW = N_EDGES // NW
CH = 80
NCHUNK = EPW // CH
NP = 10240
RPT = NP // NS
NSUB = 2
NVREG = EPW // 16

_SC_MESH = plsc.VectorSubcoreMesh(
    core_axis_name="c", subcore_axis_name="s", num_cores=NC, num_subcores=NS)


def _gather_body(x_hbm, row_hbm, col_hbm, gr_hbm, gc_hbm,
                 row_v, col_v, gr_v, gc_v, sem_r, sem_c):
    wid = lax.axis_index("s") * NC + lax.axis_index("c")
    pltpu.sync_copy(row_hbm.at[wid], row_v)
    pltpu.sync_copy(col_hbm.at[wid], col_v)

    def body(i, carry):
        base = wid * EPW + i * CH
        cpr = pltpu.async_copy(x_hbm.at[row_v.at[i]], gr_v, sem_r)
        cpc = pltpu.async_copy(x_hbm.at[col_v.at[i]], gc_v, sem_c)
        cpr.wait()
        cpc.wait()
        pltpu.sync_copy(gr_v, gr_hbm.at[pl.ds(base, CH)])
        pltpu.sync_copy(gc_v, gc_hbm.at[pl.ds(base, CH)])
        return carry

    lax.fori_loop(0, NCHUNK, body, 0)


@functools.partial(
    pl.kernel,
    out_type=[
        jax.ShapeDtypeStruct((N_EDGES, D), jnp.float32),
        jax.ShapeDtypeStruct((N_EDGES, D), jnp.float32),
    ],
    mesh=_SC_MESH,
    scratch_types=[
        pltpu.VMEM((NCHUNK, CH), jnp.int32),
        pltpu.VMEM((NCHUNK, CH), jnp.int32),
        pltpu.VMEM((CH, D), jnp.float32),
        pltpu.VMEM((CH, D), jnp.float32),
        pltpu.SemaphoreType.DMA,
        pltpu.SemaphoreType.DMA,
    ],
)
def _gather(x, row3, col3, gr, gc, *scratch):
    _gather_body(x, row3, col3, gr, gc, *scratch)


def _edge_body(gr_ref, gc_ref, ea_ref, w1a_ref, w1b_ref, w1c_ref, b1_ref,
               w2_ref, b2_ref, nw1a_ref, nw1b_ref, nb1_ref,
               nea_ref, hn_ref):
    f32 = jnp.float32
    bf16 = jnp.bfloat16
    gr = gr_ref[...].astype(bf16)
    gc = gc_ref[...].astype(bf16)
    ab = (jnp.dot(gr, w1a_ref[...], preferred_element_type=f32)
          + jnp.dot(gc, w1b_ref[...], preferred_element_type=f32)
          + jnp.dot(ea_ref[...], w1c_ref[...], preferred_element_type=f32)
          + b1_ref[...])
    he = jnp.maximum(ab, 0.0).astype(bf16)
    nea = jnp.dot(he, w2_ref[...], preferred_element_type=f32) + b2_ref[...]
    nea_ref[...] = nea
    hn_ref[...] = jnp.maximum(
        jnp.dot(gr, nw1a_ref[...], preferred_element_type=f32)
        + jnp.dot(nea.astype(bf16), nw1b_ref[...], preferred_element_type=f32)
        + nb1_ref[...], 0.0)


def _edge(gr, gc, ea, w1a, w1b, w1c, b1, w2, b2, nw1a, nw1b, nb1):
    blk = 8000

    def full(shape):
        return pl.BlockSpec(shape, lambda i: (0, 0))

    return pl.pallas_call(
        _edge_body,
        grid=(N_EDGES // blk,),
        in_specs=[
            pl.BlockSpec((blk, D), lambda i: (i, 0)),
            pl.BlockSpec((blk, D), lambda i: (i, 0)),
            pl.BlockSpec((blk, DE), lambda i: (i, 0)),
            full((D, DE)),
            full((D, DE)),
            full((DE, DE)),
            full((1, DE)),
            full((DE, DE)),
            full((1, DE)),
            full((D, D)),
            full((DE, D)),
            full((1, D)),
        ],
        out_specs=[
            pl.BlockSpec((blk, DE), lambda i: (i, 0)),
            pl.BlockSpec((blk, D), lambda i: (i, 0)),
        ],
        out_shape=[
            jax.ShapeDtypeStruct((N_EDGES, DE), jnp.float32),
            jax.ShapeDtypeStruct((N_EDGES, D), jnp.float32),
        ],
    )(gr, gc, ea, w1a, w1b, w1c, b1, w2, b2, nw1a, nw1b, nb1)


def _scatter_body(hn_hbm, col_hbm, zacc_hbm, zcnt_hbm,
                  part_hbm, cnt_hbm,
                  col_v, hn_v, cnt8_v, acc):
    c_id = lax.axis_index("c")
    s_id = lax.axis_index("s")
    wid = s_id * NC + c_id
    rbase = s_id * RPT
    pltpu.sync_copy(zacc_hbm.at[pl.ds(rbase, RPT)], acc.at[pl.ds(rbase, RPT)])
    pltpu.sync_copy(zcnt_hbm, cnt8_v)
    pltpu.sync_copy(col_hbm.at[wid], col_v)
    plsc.subcore_barrier()

    def body(i, carry):
        base = wid * EPW + i * CH
        pltpu.sync_copy(hn_hbm.at[pl.ds(base, CH)], hn_v)
        pltpu.sync_copy(hn_v, acc.at[col_v.at[i]], add=True)
        return carry

    lax.fori_loop(0, NCHUNK, body, 0)

    lane = lax.iota(jnp.int32, 16)
    ioff = lax.rem(lane, NSUB) * NP
    group = lane // NSUB
    masks = [group == g for g in range(16 // NSUB)]
    ones16 = jnp.full((16,), 1.0, jnp.float32)
    npair = CH // 16

    def cbody(k, carry):
        i = k // npair
        j = k - i * npair
        idx = col_v[i, pl.ds(j * 16, 16)] + ioff
        for m in masks:
            plsc.addupdate_scatter(cnt8_v, [idx], ones16, mask=m)
        return carry

    lax.fori_loop(0, NVREG, cbody, 0)
    plsc.subcore_barrier()
    pltpu.sync_copy(acc.at[pl.ds(rbase, RPT)],
                    part_hbm.at[c_id].at[pl.ds(rbase, RPT)])
    pltpu.sync_copy(cnt8_v, cnt_hbm.at[wid])


@functools.partial(
    pl.kernel,
    out_type=[
        jax.ShapeDtypeStruct((NC, NP, D), jnp.float32),
        jax.ShapeDtypeStruct((NW, NSUB * NP), jnp.float32),
    ],
    mesh=_SC_MESH,
    scratch_types=[
        pltpu.VMEM((NCHUNK, CH), jnp.int32),
        pltpu.VMEM((CH, D), jnp.float32),
        pltpu.VMEM((NSUB * NP,), jnp.float32),
        pltpu.VMEM_SHARED((NP, D), jnp.float32),
    ],
    compiler_params=pltpu.CompilerParams(needs_layout_passes=False),
)
def _scatter(hn, col3, zacc, zcnt, part, cnt, *scratch):
    _scatter_body(hn, col3, zacc, zcnt, part, cnt, *scratch)


def _post_body(p0_ref, p1_ref, c_ref, ones_ref, w_ref, nb2_ref, out_ref):
    sums = p0_ref[...] + p1_ref[...]
    cnt = lax.dot_general(c_ref[...], ones_ref[...],
                           (((0,), (0,)), ((), ())),
                           preferred_element_type=jnp.float32)
    denom = jnp.maximum(cnt, 1.0)
    out_ref[...] = jnp.maximum(
        (jnp.dot(sums, w_ref[...], preferred_element_type=jnp.float32)
         + nb2_ref[...] * cnt) / denom, 0.0)


def _post(p0, p1, cnt_t, ones, w, nb2):
    blk = 1280
    nsh = NW * NSUB
    return pl.pallas_call(
        _post_body,
        grid=(NP // blk,),
        in_specs=[
            pl.BlockSpec((blk, D), lambda i: (i, 0)),
            pl.BlockSpec((blk, D), lambda i: (i, 0)),
            pl.BlockSpec((nsh, blk), lambda i: (0, i)),
            pl.BlockSpec((nsh, 1), lambda i: (0, 0)),
            pl.BlockSpec((D, D), lambda i: (0, 0)),
            pl.BlockSpec((1, D), lambda i: (0, 0)),
        ],
        out_specs=pl.BlockSpec((blk, D), lambda i: (i, 0)),
        out_shape=jax.ShapeDtypeStruct((NP, D), jnp.float32),
    )(p0, p1, cnt_t, ones, w, nb2)


def kernel(x, edge_index, edge_attr, eW1, eb1, eW2, eb2, nW1, nb1, nW2, nb2):
    row = edge_index[0].astype(jnp.int32)
    col = edge_index[1].astype(jnp.int32)
    row3 = row.reshape(NW, NCHUNK, CH)
    col3 = col.reshape(NW, NCHUNK, CH)
    bf16 = jnp.bfloat16
    gr, gc = _gather(x, row3, col3)

    nea, hn = _edge(gr, gc, edge_attr.astype(bf16),
                    eW1[:D].astype(bf16), eW1[D:2 * D].astype(bf16),
                    eW1[2 * D:].astype(bf16), eb1.reshape(1, DE),
                    eW2.astype(bf16), eb2.reshape(1, DE),
                    nW1[:D].astype(bf16), nW1[D:].astype(bf16),
                    nb1.reshape(1, D))

    zacc = jnp.zeros((NP, D), jnp.float32)
    zcnt = jnp.zeros((NSUB * NP,), jnp.float32)
    part, cnt = _scatter(hn, col3, zacc, zcnt)
    cnt_t = cnt.reshape(NW * NSUB, NP)
    ones = jnp.ones((NW * NSUB, 1), jnp.float32)
    out = _post(part[0], part[1], cnt_t, ones, nW2, nb2.reshape(1, D))
    return out[:N_NODES], nea


# speedup vs baseline: 1.2122x; 1.0511x over previous
import functools

import jax
import jax.numpy as jnp
from jax import lax
from jax.experimental import pallas as pl
from jax.experimental.pallas import tpu as pltpu
from jax.experimental.pallas import tpu_sc as plsc

N_NODES = 10000
N_EDGES = 320000
D = 128
DE = 16

NC = 2
NS = 16
NW = NC * NS
EPW = N_EDGES // NW
CH = 80
NCHUNK = EPW // CH
NP = 10240
RPT = NP // NS
NSUB = 2
NVREG = EPW // 16

_SC_MESH = plsc.VectorSubcoreMesh(
    core_axis_name="c", subcore_axis_name="s", num_cores=NC, num_subcores=NS)


NPAIR = (NCHUNK - 1) // 2


def _gather_body(x_hbm, row_hbm, col_hbm, gr_hbm, gc_hbm,
                 row_v, col_v, gr0, gc0, gr1, gc1, si0, si1, so0, so1):
    wid = lax.axis_index("s") * NC + lax.axis_index("c")
    pltpu.sync_copy(row_hbm.at[wid], row_v)
    pltpu.sync_copy(col_hbm.at[wid], col_v)
    bufs = ((gr0, gc0, si0, so0), (gr1, gc1, si1, so1))

    def issue_in(i, b):
        gr_v, gc_v, si, _ = bufs[b]
        pltpu.async_copy(x_hbm.at[row_v.at[i]], gr_v, si)
        pltpu.async_copy(x_hbm.at[col_v.at[i]], gc_v, si)

    def wait_in(b):
        gr_v, gc_v, si, _ = bufs[b]
        pltpu.make_async_copy(x_hbm.at[pl.ds(0, CH)], gr_v, si).wait()
        pltpu.make_async_copy(x_hbm.at[pl.ds(0, CH)], gc_v, si).wait()

    def issue_out(i, b):
        gr_v, gc_v, _, so = bufs[b]
        base = wid * EPW + i * CH
        pltpu.async_copy(gr_v, gr_hbm.at[pl.ds(base, CH)], so)
        pltpu.async_copy(gc_v, gc_hbm.at[pl.ds(base, CH)], so)

    def wait_out(b):
        gr_v, gc_v, _, so = bufs[b]
        pltpu.make_async_copy(gr_v, gr_hbm.at[pl.ds(0, CH)], so).wait()
        pltpu.make_async_copy(gc_v, gc_hbm.at[pl.ds(0, CH)], so).wait()

    issue_in(0, 0)
    issue_in(1, 1)

    def body(g, carry):
        i0 = g * 2
        wait_in(0)
        issue_out(i0, 0)
        wait_in(1)
        issue_out(i0 + 1, 1)
        wait_out(0)

        @pl.when(g < NPAIR - 1)
        def _():
            issue_in(i0 + 2, 0)

        wait_out(1)

        @pl.when(g < NPAIR - 1)
        def _():
            issue_in(i0 + 3, 1)

        return carry

    lax.fori_loop(0, NPAIR, body, 0)

    i = NCHUNK - 1
    issue_in(i, 0)
    wait_in(0)
    issue_out(i, 0)
    wait_out(0)


@functools.partial(
    pl.kernel,
    out_type=[
        jax.ShapeDtypeStruct((N_EDGES, D), jnp.float32),
        jax.ShapeDtypeStruct((N_EDGES, D), jnp.float32),
    ],
    mesh=_SC_MESH,
    scratch_types=[
        pltpu.VMEM((NCHUNK, CH), jnp.int32),
        pltpu.VMEM((NCHUNK, CH), jnp.int32),
        pltpu.VMEM((CH, D), jnp.float32),
        pltpu.VMEM((CH, D), jnp.float32),
        pltpu.VMEM((CH, D), jnp.float32),
        pltpu.VMEM((CH, D), jnp.float32),
        pltpu.SemaphoreType.DMA,
        pltpu.SemaphoreType.DMA,
        pltpu.SemaphoreType.DMA,
        pltpu.SemaphoreType.DMA,
    ],
)
def _gather(x, row3, col3, gr, gc, *scratch):
    _gather_body(x, row3, col3, gr, gc, *scratch)


def _edge_body(gr_ref, gc_ref, ea_ref, w1a_ref, w1b_ref, w1c_ref, b1_ref,
               w2_ref, b2_ref, nw1a_ref, nw1b_ref, nb1_ref,
               nea_ref, hn_ref):
    f32 = jnp.float32
    bf16 = jnp.bfloat16
    gr = gr_ref[...].astype(bf16)
    gc = gc_ref[...].astype(bf16)
    ab = (jnp.dot(gr, w1a_ref[...], preferred_element_type=f32)
          + jnp.dot(gc, w1b_ref[...], preferred_element_type=f32)
          + jnp.dot(ea_ref[...], w1c_ref[...], preferred_element_type=f32)
          + b1_ref[...])
    he = jnp.maximum(ab, 0.0).astype(bf16)
    nea = jnp.dot(he, w2_ref[...], preferred_element_type=f32) + b2_ref[...]
    nea_ref[...] = nea
    hn_ref[...] = jnp.maximum(
        jnp.dot(gr, nw1a_ref[...], preferred_element_type=f32)
        + jnp.dot(nea.astype(bf16), nw1b_ref[...], preferred_element_type=f32)
        + nb1_ref[...], 0.0)


def _edge(gr, gc, ea, w1a, w1b, w1c, b1, w2, b2, nw1a, nw1b, nb1):
    blk = 8000

    def full(shape):
        return pl.BlockSpec(shape, lambda i: (0, 0))

    return pl.pallas_call(
        _edge_body,
        grid=(N_EDGES // blk,),
        in_specs=[
            pl.BlockSpec((blk, D), lambda i: (i, 0)),
            pl.BlockSpec((blk, D), lambda i: (i, 0)),
            pl.BlockSpec((blk, DE), lambda i: (i, 0)),
            full((D, DE)),
            full((D, DE)),
            full((DE, DE)),
            full((1, DE)),
            full((DE, DE)),
            full((1, DE)),
            full((D, D)),
            full((DE, D)),
            full((1, D)),
        ],
        out_specs=[
            pl.BlockSpec((blk, DE), lambda i: (i, 0)),
            pl.BlockSpec((blk, D), lambda i: (i, 0)),
        ],
        out_shape=[
            jax.ShapeDtypeStruct((N_EDGES, DE), jnp.float32),
            jax.ShapeDtypeStruct((N_EDGES, D), jnp.float32),
        ],
    )(gr, gc, ea, w1a, w1b, w1c, b1, w2, b2, nw1a, nw1b, nb1)


def _scatter_body(hn_hbm, col_hbm, zacc_hbm, zcnt_hbm,
                  part_hbm, cnt_hbm,
                  col_v, hn_v, cnt8_v, acc):
    c_id = lax.axis_index("c")
    s_id = lax.axis_index("s")
    wid = s_id * NC + c_id
    rbase = s_id * RPT
    pltpu.sync_copy(zacc_hbm.at[pl.ds(rbase, RPT)], acc.at[pl.ds(rbase, RPT)])
    pltpu.sync_copy(zcnt_hbm, cnt8_v)
    pltpu.sync_copy(col_hbm.at[wid], col_v)
    plsc.subcore_barrier()

    def body(i, carry):
        base = wid * EPW + i * CH
        pltpu.sync_copy(hn_hbm.at[pl.ds(base, CH)], hn_v)
        pltpu.sync_copy(hn_v, acc.at[col_v.at[i]], add=True)
        return carry

    lax.fori_loop(0, NCHUNK, body, 0)

    lane = lax.iota(jnp.int32, 16)
    ioff = lax.rem(lane, NSUB) * NP
    group = lane // NSUB
    masks = [group == g for g in range(16 // NSUB)]
    ones16 = jnp.full((16,), 1.0, jnp.float32)
    npair = CH // 16

    def cbody(k, carry):
        i = k // npair
        j = k - i * npair
        idx = col_v[i, pl.ds(j * 16, 16)] + ioff
        for m in masks:
            plsc.addupdate_scatter(cnt8_v, [idx], ones16, mask=m)
        return carry

    lax.fori_loop(0, NVREG, cbody, 0)
    plsc.subcore_barrier()
    pltpu.sync_copy(acc.at[pl.ds(rbase, RPT)],
                    part_hbm.at[c_id].at[pl.ds(rbase, RPT)])
    pltpu.sync_copy(cnt8_v, cnt_hbm.at[wid])


@functools.partial(
    pl.kernel,
    out_type=[
        jax.ShapeDtypeStruct((NC, NP, D), jnp.float32),
        jax.ShapeDtypeStruct((NW, NSUB * NP), jnp.float32),
    ],
    mesh=_SC_MESH,
    scratch_types=[
        pltpu.VMEM((NCHUNK, CH), jnp.int32),
        pltpu.VMEM((CH, D), jnp.float32),
        pltpu.VMEM((NSUB * NP,), jnp.float32),
        pltpu.VMEM_SHARED((NP, D), jnp.float32),
    ],
    compiler_params=pltpu.CompilerParams(needs_layout_passes=False),
)
def _scatter(hn, col3, zacc, zcnt, part, cnt, *scratch):
    _scatter_body(hn, col3, zacc, zcnt, part, cnt, *scratch)


def _post_body(p0_ref, p1_ref, c_ref, ones_ref, w_ref, nb2_ref, out_ref):
    sums = p0_ref[...] + p1_ref[...]
    cnt = lax.dot_general(c_ref[...], ones_ref[...],
                           (((0,), (0,)), ((), ())),
                           preferred_element_type=jnp.float32)
    denom = jnp.maximum(cnt, 1.0)
    out_ref[...] = jnp.maximum(
        (jnp.dot(sums, w_ref[...], preferred_element_type=jnp.float32)
         + nb2_ref[...] * cnt) / denom, 0.0)


def _post(p0, p1, cnt_t, ones, w, nb2):
    blk = 1280
    nsh = NW * NSUB
    return pl.pallas_call(
        _post_body,
        grid=(NP // blk,),
        in_specs=[
            pl.BlockSpec((blk, D), lambda i: (i, 0)),
            pl.BlockSpec((blk, D), lambda i: (i, 0)),
            pl.BlockSpec((nsh, blk), lambda i: (0, i)),
            pl.BlockSpec((nsh, 1), lambda i: (0, 0)),
            pl.BlockSpec((D, D), lambda i: (0, 0)),
            pl.BlockSpec((1, D), lambda i: (0, 0)),
        ],
        out_specs=pl.BlockSpec((blk, D), lambda i: (i, 0)),
        out_shape=jax.ShapeDtypeStruct((NP, D), jnp.float32),
    )(p0, p1, cnt_t, ones, w, nb2)


def kernel(x, edge_index, edge_attr, eW1, eb1, eW2, eb2, nW1, nb1, nW2, nb2):
    row = edge_index[0].astype(jnp.int32)
    col = edge_index[1].astype(jnp.int32)
    row3 = row.reshape(NW, NCHUNK, CH)
    col3 = col.reshape(NW, NCHUNK, CH)
    bf16 = jnp.bfloat16
    gr, gc = _gather(x, row3, col3)

    nea, hn = _edge(gr, gc, edge_attr.astype(bf16),
                    eW1[:D].astype(bf16), eW1[D:2 * D].astype(bf16),
                    eW1[2 * D:].astype(bf16), eb1.reshape(1, DE),
                    eW2.astype(bf16), eb2.reshape(1, DE),
                    nW1[:D].astype(bf16), nW1[D:].astype(bf16),
                    nb1.reshape(1, D))

    zacc = jnp.zeros((NP, D), jnp.float32)
    zcnt = jnp.zeros((NSUB * NP,), jnp.float32)
    part, cnt = _scatter(hn, col3, zacc, zcnt)
    cnt_t = cnt.reshape(NW * NSUB, NP)
    ones = jnp.ones((NW * NSUB, 1), jnp.float32)
    out = _post(part[0], part[1], cnt_t, ones, nW2, nb2.reshape(1, D))
    return out[:N_NODES], nea


# speedup vs baseline: 1.3476x; 1.1117x over previous
import functools

import jax
import jax.numpy as jnp
from jax import lax
from jax.experimental import pallas as pl
from jax.experimental.pallas import tpu as pltpu
from jax.experimental.pallas import tpu_sc as plsc

N_NODES = 10000
N_EDGES = 320000
D = 128
DE = 16

NC = 2
NS = 16
NW = NC * NS
EPW = N_EDGES // NW
CH = 80
NCHUNK = EPW // CH
NP = 10240
RPT = NP // NS
NSUB = 1
NVREG = EPW // 16

_SC_MESH = plsc.VectorSubcoreMesh(
    core_axis_name="c", subcore_axis_name="s", num_cores=NC, num_subcores=NS)


NPAIR = (NCHUNK - 1) // 2


def _gather_body(x_hbm, row_hbm, col_hbm, gr_hbm, gc_hbm,
                 row_v, col_v, gr0, gc0, gr1, gc1, si0, si1, so0, so1):
    wid = lax.axis_index("s") * NC + lax.axis_index("c")
    pltpu.sync_copy(row_hbm.at[wid], row_v)
    pltpu.sync_copy(col_hbm.at[wid], col_v)
    bufs = ((gr0, gc0, si0, so0), (gr1, gc1, si1, so1))

    def issue_in(i, b):
        gr_v, gc_v, si, _ = bufs[b]
        pltpu.async_copy(x_hbm.at[row_v.at[i]], gr_v, si)
        pltpu.async_copy(x_hbm.at[col_v.at[i]], gc_v, si)

    def wait_in(b):
        gr_v, gc_v, si, _ = bufs[b]
        pltpu.make_async_copy(x_hbm.at[pl.ds(0, CH)], gr_v, si).wait()
        pltpu.make_async_copy(x_hbm.at[pl.ds(0, CH)], gc_v, si).wait()

    def issue_out(i, b):
        gr_v, gc_v, _, so = bufs[b]
        base = wid * EPW + i * CH
        pltpu.async_copy(gr_v, gr_hbm.at[pl.ds(base, CH)], so)
        pltpu.async_copy(gc_v, gc_hbm.at[pl.ds(base, CH)], so)

    def wait_out(b):
        gr_v, gc_v, _, so = bufs[b]
        pltpu.make_async_copy(gr_v, gr_hbm.at[pl.ds(0, CH)], so).wait()
        pltpu.make_async_copy(gc_v, gc_hbm.at[pl.ds(0, CH)], so).wait()

    issue_in(0, 0)
    issue_in(1, 1)

    def body(g, carry):
        i0 = g * 2
        wait_in(0)
        issue_out(i0, 0)
        wait_in(1)
        issue_out(i0 + 1, 1)
        wait_out(0)

        @pl.when(g < NPAIR - 1)
        def _():
            issue_in(i0 + 2, 0)

        wait_out(1)

        @pl.when(g < NPAIR - 1)
        def _():
            issue_in(i0 + 3, 1)

        return carry

    lax.fori_loop(0, NPAIR, body, 0)

    i = NCHUNK - 1
    issue_in(i, 0)
    wait_in(0)
    issue_out(i, 0)
    wait_out(0)


@functools.partial(
    pl.kernel,
    out_type=[
        jax.ShapeDtypeStruct((N_EDGES, D), jnp.float32),
        jax.ShapeDtypeStruct((N_EDGES, D), jnp.float32),
    ],
    mesh=_SC_MESH,
    scratch_types=[
        pltpu.VMEM((NCHUNK, CH), jnp.int32),
        pltpu.VMEM((NCHUNK, CH), jnp.int32),
        pltpu.VMEM((CH, D), jnp.float32),
        pltpu.VMEM((CH, D), jnp.float32),
        pltpu.VMEM((CH, D), jnp.float32),
        pltpu.VMEM((CH, D), jnp.float32),
        pltpu.SemaphoreType.DMA,
        pltpu.SemaphoreType.DMA,
        pltpu.SemaphoreType.DMA,
        pltpu.SemaphoreType.DMA,
    ],
)
def _gather(x, row3, col3, gr, gc, *scratch):
    _gather_body(x, row3, col3, gr, gc, *scratch)


def _edge_body(gr_ref, gc_ref, ea_ref, w1a_ref, w1b_ref, w1c_ref, b1_ref,
               w2_ref, b2_ref, nw1a_ref, nw1b_ref, nb1_ref,
               nea_ref, hn_ref):
    f32 = jnp.float32
    bf16 = jnp.bfloat16
    gr = gr_ref[...].astype(bf16)
    gc = gc_ref[...].astype(bf16)
    ab = (jnp.dot(gr, w1a_ref[...], preferred_element_type=f32)
          + jnp.dot(gc, w1b_ref[...], preferred_element_type=f32)
          + jnp.dot(ea_ref[...], w1c_ref[...], preferred_element_type=f32)
          + b1_ref[...])
    he = jnp.maximum(ab, 0.0).astype(bf16)
    nea = jnp.dot(he, w2_ref[...], preferred_element_type=f32) + b2_ref[...]
    nea_ref[...] = nea
    hn_ref[...] = jnp.maximum(
        jnp.dot(gr, nw1a_ref[...], preferred_element_type=f32)
        + jnp.dot(nea.astype(bf16), nw1b_ref[...], preferred_element_type=f32)
        + nb1_ref[...], 0.0)


def _edge(gr, gc, ea, w1a, w1b, w1c, b1, w2, b2, nw1a, nw1b, nb1):
    blk = 8000

    def full(shape):
        return pl.BlockSpec(shape, lambda i: (0, 0))

    return pl.pallas_call(
        _edge_body,
        grid=(N_EDGES // blk,),
        in_specs=[
            pl.BlockSpec((blk, D), lambda i: (i, 0)),
            pl.BlockSpec((blk, D), lambda i: (i, 0)),
            pl.BlockSpec((blk, DE), lambda i: (i, 0)),
            full((D, DE)),
            full((D, DE)),
            full((DE, DE)),
            full((1, DE)),
            full((DE, DE)),
            full((1, DE)),
            full((D, D)),
            full((DE, D)),
            full((1, D)),
        ],
        out_specs=[
            pl.BlockSpec((blk, DE), lambda i: (i, 0)),
            pl.BlockSpec((blk, D), lambda i: (i, 0)),
        ],
        out_shape=[
            jax.ShapeDtypeStruct((N_EDGES, DE), jnp.float32),
            jax.ShapeDtypeStruct((N_EDGES, D), jnp.float32),
        ],
    )(gr, gc, ea, w1a, w1b, w1c, b1, w2, b2, nw1a, nw1b, nb1)


def _scatter_body(hn_hbm, col_hbm, zacc_hbm, zcnt_hbm,
                  part_hbm, cnt_hbm,
                  col_v, hn0, hn1, sl0, sl1, cnt8_v, acc):
    c_id = lax.axis_index("c")
    s_id = lax.axis_index("s")
    wid = s_id * NC + c_id
    rbase = s_id * RPT
    pltpu.sync_copy(zacc_hbm.at[pl.ds(rbase, RPT)], acc.at[pl.ds(rbase, RPT)])
    pltpu.sync_copy(zcnt_hbm, cnt8_v)
    pltpu.sync_copy(col_hbm.at[wid], col_v)
    plsc.subcore_barrier()
    bufs = ((hn0, sl0), (hn1, sl1))

    def issue_load(i, b):
        hn_v, sl = bufs[b]
        base = wid * EPW + i * CH
        pltpu.async_copy(hn_hbm.at[pl.ds(base, CH)], hn_v, sl)

    def wait_load(b):
        hn_v, sl = bufs[b]
        pltpu.make_async_copy(hn_hbm.at[pl.ds(0, CH)], hn_v, sl).wait()

    def add(i, b):
        hn_v, _ = bufs[b]
        pltpu.sync_copy(hn_v, acc.at[col_v.at[i]], add=True)

    issue_load(0, 0)
    issue_load(1, 1)

    def body(g, carry):
        i0 = g * 2
        wait_load(0)
        add(i0, 0)

        @pl.when(g < NPAIR - 1)
        def _():
            issue_load(i0 + 2, 0)

        wait_load(1)
        add(i0 + 1, 1)

        @pl.when(g < NPAIR - 1)
        def _():
            issue_load(i0 + 3, 1)

        return carry

    lax.fori_loop(0, NPAIR, body, 0)

    i = NCHUNK - 1
    issue_load(i, 0)
    wait_load(0)
    add(i, 0)

    lane = lax.iota(jnp.int32, 16)
    ioff = lax.rem(lane, NSUB) * NP
    group = lane // NSUB
    masks = [group == g for g in range(16 // NSUB)]
    ones16 = jnp.full((16,), 1.0, jnp.float32)
    npair = CH // 16

    def cbody(k, carry):
        i = k // npair
        j = k - i * npair
        idx = col_v[i, pl.ds(j * 16, 16)] + ioff
        for m in masks:
            plsc.addupdate_scatter(cnt8_v, [idx], ones16, mask=m)
        return carry

    lax.fori_loop(0, NVREG, cbody, 0)
    plsc.subcore_barrier()
    pltpu.sync_copy(acc.at[pl.ds(rbase, RPT)],
                    part_hbm.at[c_id].at[pl.ds(rbase, RPT)])
    pltpu.sync_copy(cnt8_v, cnt_hbm.at[wid])


@functools.partial(
    pl.kernel,
    out_type=[
        jax.ShapeDtypeStruct((NC, NP, D), jnp.float32),
        jax.ShapeDtypeStruct((NW, NSUB * NP), jnp.float32),
    ],
    mesh=_SC_MESH,
    scratch_types=[
        pltpu.VMEM((NCHUNK, CH), jnp.int32),
        pltpu.VMEM((CH, D), jnp.float32),
        pltpu.VMEM((CH, D), jnp.float32),
        pltpu.SemaphoreType.DMA,
        pltpu.SemaphoreType.DMA,
        pltpu.VMEM((NSUB * NP,), jnp.float32),
        pltpu.VMEM_SHARED((NP, D), jnp.float32),
    ],
    compiler_params=pltpu.CompilerParams(needs_layout_passes=False),
)
def _scatter(hn, col3, zacc, zcnt, part, cnt, *scratch):
    _scatter_body(hn, col3, zacc, zcnt, part, cnt, *scratch)


def _post_body(p0_ref, p1_ref, c_ref, ones_ref, w_ref, nb2_ref, out_ref):
    sums = p0_ref[...] + p1_ref[...]
    cnt = lax.dot_general(c_ref[...], ones_ref[...],
                           (((0,), (0,)), ((), ())),
                           preferred_element_type=jnp.float32)
    denom = jnp.maximum(cnt, 1.0)
    out_ref[...] = jnp.maximum(
        (jnp.dot(sums, w_ref[...], preferred_element_type=jnp.float32)
         + nb2_ref[...] * cnt) / denom, 0.0)


def _post(p0, p1, cnt_t, ones, w, nb2):
    blk = 1280
    nsh = NW * NSUB
    return pl.pallas_call(
        _post_body,
        grid=(NP // blk,),
        in_specs=[
            pl.BlockSpec((blk, D), lambda i: (i, 0)),
            pl.BlockSpec((blk, D), lambda i: (i, 0)),
            pl.BlockSpec((nsh, blk), lambda i: (0, i)),
            pl.BlockSpec((nsh, 1), lambda i: (0, 0)),
            pl.BlockSpec((D, D), lambda i: (0, 0)),
            pl.BlockSpec((1, D), lambda i: (0, 0)),
        ],
        out_specs=pl.BlockSpec((blk, D), lambda i: (i, 0)),
        out_shape=jax.ShapeDtypeStruct((NP, D), jnp.float32),
    )(p0, p1, cnt_t, ones, w, nb2)


def kernel(x, edge_index, edge_attr, eW1, eb1, eW2, eb2, nW1, nb1, nW2, nb2):
    row = edge_index[0].astype(jnp.int32)
    col = edge_index[1].astype(jnp.int32)
    row3 = row.reshape(NW, NCHUNK, CH)
    col3 = col.reshape(NW, NCHUNK, CH)
    bf16 = jnp.bfloat16
    gr, gc = _gather(x, row3, col3)

    nea, hn = _edge(gr, gc, edge_attr.astype(bf16),
                    eW1[:D].astype(bf16), eW1[D:2 * D].astype(bf16),
                    eW1[2 * D:].astype(bf16), eb1.reshape(1, DE),
                    eW2.astype(bf16), eb2.reshape(1, DE),
                    nW1[:D].astype(bf16), nW1[D:].astype(bf16),
                    nb1.reshape(1, D))

    zacc = jnp.zeros((NP, D), jnp.float32)
    zcnt = jnp.zeros((NSUB * NP,), jnp.float32)
    part, cnt = _scatter(hn, col3, zacc, zcnt)
    cnt_t = cnt.reshape(NW * NSUB, NP)
    ones = jnp.ones((NW * NSUB, 1), jnp.float32)
    out = _post(part[0], part[1], cnt_t, ones, nW2, nb2.reshape(1, D))
    return out[:N_NODES], nea
